# baseline (speedup 1.0000x reference)
import functools

import numpy as np

import jax
import jax.numpy as jnp
from jax import lax
from jax.experimental import pallas as pl
from jax.experimental.pallas import tpu as pltpu
from jax.experimental.pallas import tpu_sc as plsc

N = 10000
NPAD = 10240
E = 320000
EPAD = 327680
D = 128
NC, NS = 2, 16
EPT = EPAD // (NC * NS)
CH = 128
NCHUNK = EPT // CH
RPT = NPAD // NS
HALF = NCHUNK // 2
NCH0, NCH1 = 40, 120
NPHASE = (max(NCH0, NCH1) + HALF - 1) // HALF

BN = 400
NBLK = N // BN

_mesh = plsc.VectorSubcoreMesh(core_axis_name="c", subcore_axis_name="s")


def _zero_acc(acc, rows, s):
    def body(k, _):
        pltpu.sync_copy(rows, acc.at[pl.ds(s * RPT + k * CH, CH)])
        return _

    lax.fori_loop(jnp.int32(0), jnp.int32(RPT // CH), body, None)


def _writeback(acc, rows, out_hbm, c, s):
    def body(k, _):
        r = s * RPT + k * CH
        pltpu.sync_copy(acc.at[pl.ds(r, CH)], rows)
        pltpu.sync_copy(rows, out_hbm.at[c, pl.ds(r, CH)])
        return _

    lax.fori_loop(jnp.int32(0), jnp.int32(RPT // CH), body, None)


def _deg_body(dst_hbm, one_hbm, zer_hbm, out_hbm, acc, dstbuf, zrows, orows):
    c = lax.axis_index("c")
    s = lax.axis_index("s")
    g = c * NS + s
    pltpu.sync_copy(zer_hbm, zrows)
    pltpu.sync_copy(one_hbm, orows)
    _zero_acc(acc, zrows, s)
    plsc.subcore_barrier()

    def chunk(i, _):
        pltpu.sync_copy(orows, acc.at[dstbuf.at[i]], add=True)
        return _

    for p in range(2):
        pltpu.sync_copy(dst_hbm.at[pl.ds(g * NCHUNK + p * HALF, HALF)], dstbuf)
        lax.fori_loop(jnp.int32(0), jnp.int32(HALF), chunk, None)
    plsc.subcore_barrier()
    _writeback(acc, zrows, out_hbm, c, s)


_deg_kernel = functools.partial(
    pl.kernel,
    out_type=jax.ShapeDtypeStruct((NC, NPAD, D), jnp.float32),
    mesh=_mesh,
    scratch_types=[
        pltpu.VMEM_SHARED((NPAD, D), jnp.float32),
        pltpu.VMEM((HALF, CH), jnp.int32),
        pltpu.VMEM((CH, D), jnp.float32),
        pltpu.VMEM((CH, D), jnp.float32),
    ],
)(_deg_body)


def _agg_body(h_hbm, src_hbm, dst_hbm, zer_hbm, out_hbm, acc,
              srcbuf, dstbuf, rows0, rows1, sem0, sem1):
    c = lax.axis_index("c")
    s = lax.axis_index("s")
    nch = jnp.where(c == 0, np.int32(NCH0), np.int32(NCH1))
    base_chunk = jnp.where(c == 0, s * np.int32(NCH0), np.int32(NS * NCH0) + s * np.int32(NCH1))
    pltpu.sync_copy(zer_hbm, rows0)
    _zero_acc(acc, rows0, s)
    plsc.subcore_barrier()

    def start_gather(i, rows, sem):
        pltpu.async_copy(h_hbm.at[srcbuf.at[i]], rows, sem)

    def wait_scatter(i, rows, sem):
        pltpu.make_async_copy(h_hbm.at[srcbuf.at[i]], rows, sem).wait()
        pltpu.sync_copy(rows, acc.at[dstbuf.at[i]], add=True)

    def body(j, _):
        i0 = 2 * j
        start_gather(i0 + 1, rows1, sem1)
        wait_scatter(i0, rows0, sem0)

        @pl.when(j < HALF // 2 - 1)
        def _():
            start_gather(i0 + 2, rows0, sem0)

        wait_scatter(i0 + 1, rows1, sem1)
        return _

    for p in range(NPHASE):
        @pl.when(jnp.int32(p * HALF) < nch)
        def _(p=p):
            b = base_chunk + p * HALF
            pltpu.sync_copy(src_hbm.at[pl.ds(b, HALF)], srcbuf)
            pltpu.sync_copy(dst_hbm.at[pl.ds(b, HALF)], dstbuf)
            start_gather(jnp.int32(0), rows0, sem0)
            lax.fori_loop(jnp.int32(0), jnp.int32(HALF // 2), body, None)
    plsc.subcore_barrier()
    _writeback(acc, rows0, out_hbm, c, s)


_agg_kernel = functools.partial(
    pl.kernel,
    out_type=jax.ShapeDtypeStruct((NC, NPAD, D), jnp.float32),
    mesh=_mesh,
    scratch_types=[
        pltpu.VMEM_SHARED((NPAD, D), jnp.float32),
        pltpu.VMEM((HALF, CH), jnp.int32),
        pltpu.VMEM((HALF, CH), jnp.int32),
        pltpu.VMEM((CH, D), jnp.float32),
        pltpu.VMEM((CH, D), jnp.float32),
        pltpu.SemaphoreType.DMA,
        pltpu.SemaphoreType.DMA,
    ],
)(_agg_body)


_I0 = np.int32(0)


def _row_spec(w):
    return pl.BlockSpec((BN, w), lambda i: (i, _I0))


def _full_spec(shape):
    nd = len(shape)
    return pl.BlockSpec(shape, lambda i: (_I0,) * nd)


def _part_spec(w):
    return pl.BlockSpec((2, BN, w), lambda i: (_I0, i, _I0))


def _k1_body(deg_ref, x_ref, w1_ref, u1_ref, dis_ref):
    deg = deg_ref[0, :, 0:1] + deg_ref[1, :, 0:1] + 1.0
    dis = lax.rsqrt(deg)
    m = lax.dot_general(x_ref[...], w1_ref[...], (((1,), (1,)), ((), ())),
                        preferred_element_type=jnp.float32)
    u1_ref[...] = dis * m
    dis_ref[...] = dis


def _k1(degp, x, W1):
    return pl.pallas_call(
        _k1_body,
        grid=(NBLK,),
        in_specs=[
            _part_spec(D),
            _row_spec(D),
            _full_spec((128, 128)),
        ],
        out_specs=[_row_spec(D), _row_spec(1)],
        out_shape=[
            jax.ShapeDtypeStruct((N, D), jnp.float32),
            jax.ShapeDtypeStruct((N, 1), jnp.float32),
        ],
    )(degp, x, W1)


def _k2_body(p_ref, u1_ref, dis_ref, b1_ref, u2_ref):
    dis = dis_ref[...]
    agg = p_ref[0] + p_ref[1] + u1_ref[...]
    h1 = jnp.maximum(dis * agg + b1_ref[...], 0.0)
    u2_ref[...] = dis * h1


def _k2(P1, u1, dis, b1):
    return pl.pallas_call(
        _k2_body,
        grid=(NBLK,),
        in_specs=[_part_spec(D), _row_spec(D), _row_spec(1), _full_spec((1, 128))],
        out_specs=_row_spec(D),
        out_shape=jax.ShapeDtypeStruct((N, D), jnp.float32),
    )(P1, u1, dis, b1)


def _k3_body(p_ref, u2_ref, dis_ref, b2_ref, w2_ref, w3_ref, u3_ref):
    dis = dis_ref[...]
    v = dis * (p_ref[0] + p_ref[1] + u2_ref[...])
    h2 = lax.dot_general(v, w2_ref[...], (((1,), (1,)), ((), ())),
                         preferred_element_type=jnp.float32)
    h2 = jnp.maximum(h2 + b2_ref[...], 0.0)
    m3 = lax.dot_general(h2, w3_ref[...], (((1,), (1,)), ((), ())),
                         preferred_element_type=jnp.float32)
    u3_ref[...] = dis * m3


def _k3(P2, u2, dis, b2, W2, W3):
    return pl.pallas_call(
        _k3_body,
        grid=(NBLK,),
        in_specs=[
            _part_spec(D), _row_spec(D), _row_spec(1),
            _full_spec((1, 256)), _full_spec((256, 128)), _full_spec((128, 256)),
        ],
        out_specs=_row_spec(D),
        out_shape=jax.ShapeDtypeStruct((N, D), jnp.float32),
    )(P2, u2, dis, b2, W2, W3)


def _k4_body(p_ref, u3_ref, dis_ref, b3_ref, wp_ref, bp_ref, out_ref, acc_ref):
    i = pl.program_id(0)
    dis = dis_ref[...]
    h3 = jnp.maximum(dis * (p_ref[0] + p_ref[1] + u3_ref[...]) + b3_ref[...], 0.0)
    csum = jnp.sum(h3, axis=0, keepdims=True)

    @pl.when(i == 0)
    def _():
        acc_ref[...] = csum

    @pl.when(i > 0)
    def _():
        acc_ref[...] = acc_ref[...] + csum

    @pl.when(i == NBLK - 1)
    def _():
        g = acc_ref[...] * (1.0 / N)
        out_ref[...] = lax.dot_general(
            g, wp_ref[...], (((1,), (1,)), ((), ())),
            preferred_element_type=jnp.float32) + bp_ref[...]


def _k4(P3, u3, dis, b3, Wp, bp):
    return pl.pallas_call(
        _k4_body,
        grid=(NBLK,),
        in_specs=[
            _part_spec(D), _row_spec(D), _row_spec(1),
            _full_spec((1, 128)), _full_spec((64, 128)), _full_spec((1, 64)),
        ],
        out_specs=_full_spec((1, 64)),
        out_shape=jax.ShapeDtypeStruct((1, 64), jnp.float32),
        scratch_shapes=[pltpu.VMEM((1, 128), jnp.float32)],
    )(P3, u3, dis, b3, Wp, bp)


def kernel(x, edge_index, W1, b1, W2, b2, W3, b3, Wp, bp):
    out_dtype = jnp.result_type(x.dtype, W1.dtype)
    f32 = jnp.float32
    x = x.astype(f32)
    W1, b1, W2, b2 = W1.astype(f32), b1.astype(f32), W2.astype(f32), b2.astype(f32)
    W3, b3, Wp, bp = W3.astype(f32), b3.astype(f32), Wp.astype(f32), bp.astype(f32)
    ei = edge_index.astype(jnp.int32)
    pad = EPAD - E
    src = jnp.concatenate([ei[0], jnp.zeros((pad,), jnp.int32)]).reshape(EPAD // CH, CH)
    dst = jnp.concatenate([ei[1], jnp.full((pad,), N, jnp.int32)]).reshape(EPAD // CH, CH)

    zerD = jnp.zeros((CH, D), f32)
    oneD = jnp.ones((CH, D), f32)

    degp = _deg_kernel(dst, oneD, zerD)
    u1, dis = _k1(degp, x, W1)
    P1 = _agg_kernel(u1, src, dst, zerD)
    u2 = _k2(P1, u1, dis, b1.reshape(1, -1))
    P2 = _agg_kernel(u2, src, dst, zerD)
    u3 = _k3(P2, u2, dis, b2.reshape(1, -1), W2, W3)
    P3 = _agg_kernel(u3, src, dst, zerD)
    out = _k4(P3, u3, dis, b3.reshape(1, -1), Wp, bp.reshape(1, -1))
    return out[0].astype(out_dtype)


# speedup vs baseline: 1.0915x; 1.0915x over previous
import functools

import numpy as np

import jax
import jax.numpy as jnp
from jax import lax
from jax.experimental import pallas as pl
from jax.experimental.pallas import tpu as pltpu
from jax.experimental.pallas import tpu_sc as plsc

N = 10000
NPAD = 10240
E = 320000
EPAD = 327680
D = 128
NC, NS = 2, 16
EPT = EPAD // (NC * NS)
CH = 128
NCHUNK = EPT // CH
RPT = NPAD // NS
HALF = NCHUNK // 2
NCH0, NCH1 = 120, 40
NPHASE = (max(NCH0, NCH1) + HALF - 1) // HALF

BN = 400
NBLK = N // BN

_mesh = plsc.VectorSubcoreMesh(core_axis_name="c", subcore_axis_name="s")


def _zero_acc(acc, rows, s):
    def body(k, _):
        pltpu.sync_copy(rows, acc.at[pl.ds(s * RPT + k * CH, CH)])
        return _

    lax.fori_loop(jnp.int32(0), jnp.int32(RPT // CH), body, None)


def _writeback(acc, rows, out_hbm, c, s):
    def body(k, _):
        r = s * RPT + k * CH
        pltpu.sync_copy(acc.at[pl.ds(r, CH)], rows)
        pltpu.sync_copy(rows, out_hbm.at[c, pl.ds(r, CH)])
        return _

    lax.fori_loop(jnp.int32(0), jnp.int32(RPT // CH), body, None)


def _deg_body(dst_hbm, one_hbm, zer_hbm, out_hbm, acc, dstbuf, zrows, orows):
    c = lax.axis_index("c")
    s = lax.axis_index("s")
    g = c * NS + s
    pltpu.sync_copy(zer_hbm, zrows)
    pltpu.sync_copy(one_hbm, orows)
    _zero_acc(acc, zrows, s)
    plsc.subcore_barrier()

    def chunk(i, _):
        pltpu.sync_copy(orows, acc.at[dstbuf.at[i]], add=True)
        return _

    for p in range(2):
        pltpu.sync_copy(dst_hbm.at[pl.ds(g * NCHUNK + p * HALF, HALF)], dstbuf)
        lax.fori_loop(jnp.int32(0), jnp.int32(HALF), chunk, None)
    plsc.subcore_barrier()
    _writeback(acc, zrows, out_hbm, c, s)


_deg_kernel = functools.partial(
    pl.kernel,
    out_type=jax.ShapeDtypeStruct((NC, NPAD, D), jnp.float32),
    mesh=_mesh,
    scratch_types=[
        pltpu.VMEM_SHARED((NPAD, D), jnp.float32),
        pltpu.VMEM((HALF, CH), jnp.int32),
        pltpu.VMEM((CH, D), jnp.float32),
        pltpu.VMEM((CH, D), jnp.float32),
    ],
)(_deg_body)


def _agg_body(h_hbm, src_hbm, dst_hbm, zer_hbm, out_hbm, acc,
              srcbuf, dstbuf, rows0, rows1, sem0, sem1):
    c = lax.axis_index("c")
    s = lax.axis_index("s")
    nch = jnp.where(c == 0, np.int32(NCH0), np.int32(NCH1))
    base_chunk = jnp.where(c == 0, s * np.int32(NCH0), np.int32(NS * NCH0) + s * np.int32(NCH1))
    pltpu.sync_copy(zer_hbm, rows0)
    _zero_acc(acc, rows0, s)
    plsc.subcore_barrier()

    def start_gather(i, rows, sem):
        pltpu.async_copy(h_hbm.at[srcbuf.at[i]], rows, sem)

    def wait_scatter(i, rows, sem):
        pltpu.make_async_copy(h_hbm.at[srcbuf.at[i]], rows, sem).wait()
        pltpu.sync_copy(rows, acc.at[dstbuf.at[i]], add=True)

    def body(j, _):
        i0 = 2 * j
        start_gather(i0 + 1, rows1, sem1)
        wait_scatter(i0, rows0, sem0)

        @pl.when(j < HALF // 2 - 1)
        def _():
            start_gather(i0 + 2, rows0, sem0)

        wait_scatter(i0 + 1, rows1, sem1)
        return _

    for p in range(NPHASE):
        @pl.when(jnp.int32(p * HALF) < nch)
        def _(p=p):
            b = base_chunk + p * HALF
            pltpu.sync_copy(src_hbm.at[pl.ds(b, HALF)], srcbuf)
            pltpu.sync_copy(dst_hbm.at[pl.ds(b, HALF)], dstbuf)
            start_gather(jnp.int32(0), rows0, sem0)
            lax.fori_loop(jnp.int32(0), jnp.int32(HALF // 2), body, None)
    plsc.subcore_barrier()
    _writeback(acc, rows0, out_hbm, c, s)


_agg_kernel = functools.partial(
    pl.kernel,
    out_type=jax.ShapeDtypeStruct((NC, NPAD, D), jnp.float32),
    mesh=_mesh,
    scratch_types=[
        pltpu.VMEM_SHARED((NPAD, D), jnp.float32),
        pltpu.VMEM((HALF, CH), jnp.int32),
        pltpu.VMEM((HALF, CH), jnp.int32),
        pltpu.VMEM((CH, D), jnp.float32),
        pltpu.VMEM((CH, D), jnp.float32),
        pltpu.SemaphoreType.DMA,
        pltpu.SemaphoreType.DMA,
    ],
)(_agg_body)


_I0 = np.int32(0)


def _row_spec(w):
    return pl.BlockSpec((BN, w), lambda i: (i, _I0))


def _full_spec(shape):
    nd = len(shape)
    return pl.BlockSpec(shape, lambda i: (_I0,) * nd)


def _part_spec(w):
    return pl.BlockSpec((2, BN, w), lambda i: (_I0, i, _I0))


def _k1_body(deg_ref, x_ref, w1_ref, u1_ref, dis_ref):
    deg = deg_ref[0, :, 0:1] + deg_ref[1, :, 0:1] + 1.0
    dis = lax.rsqrt(deg)
    m = lax.dot_general(x_ref[...], w1_ref[...], (((1,), (1,)), ((), ())),
                        preferred_element_type=jnp.float32)
    u1_ref[...] = dis * m
    dis_ref[...] = dis


def _k1(degp, x, W1):
    return pl.pallas_call(
        _k1_body,
        grid=(NBLK,),
        in_specs=[
            _part_spec(D),
            _row_spec(D),
            _full_spec((128, 128)),
        ],
        out_specs=[_row_spec(D), _row_spec(1)],
        out_shape=[
            jax.ShapeDtypeStruct((N, D), jnp.float32),
            jax.ShapeDtypeStruct((N, 1), jnp.float32),
        ],
    )(degp, x, W1)


def _k2_body(p_ref, u1_ref, dis_ref, b1_ref, u2_ref):
    dis = dis_ref[...]
    agg = p_ref[0] + p_ref[1] + u1_ref[...]
    h1 = jnp.maximum(dis * agg + b1_ref[...], 0.0)
    u2_ref[...] = dis * h1


def _k2(P1, u1, dis, b1):
    return pl.pallas_call(
        _k2_body,
        grid=(NBLK,),
        in_specs=[_part_spec(D), _row_spec(D), _row_spec(1), _full_spec((1, 128))],
        out_specs=_row_spec(D),
        out_shape=jax.ShapeDtypeStruct((N, D), jnp.float32),
    )(P1, u1, dis, b1)


def _k3_body(p_ref, u2_ref, dis_ref, b2_ref, w2_ref, w3_ref, u3_ref):
    dis = dis_ref[...]
    v = dis * (p_ref[0] + p_ref[1] + u2_ref[...])
    h2 = lax.dot_general(v, w2_ref[...], (((1,), (1,)), ((), ())),
                         preferred_element_type=jnp.float32)
    h2 = jnp.maximum(h2 + b2_ref[...], 0.0)
    m3 = lax.dot_general(h2, w3_ref[...], (((1,), (1,)), ((), ())),
                         preferred_element_type=jnp.float32)
    u3_ref[...] = dis * m3


def _k3(P2, u2, dis, b2, W2, W3):
    return pl.pallas_call(
        _k3_body,
        grid=(NBLK,),
        in_specs=[
            _part_spec(D), _row_spec(D), _row_spec(1),
            _full_spec((1, 256)), _full_spec((256, 128)), _full_spec((128, 256)),
        ],
        out_specs=_row_spec(D),
        out_shape=jax.ShapeDtypeStruct((N, D), jnp.float32),
    )(P2, u2, dis, b2, W2, W3)


def _k4_body(p_ref, u3_ref, dis_ref, b3_ref, wp_ref, bp_ref, out_ref, acc_ref):
    i = pl.program_id(0)
    dis = dis_ref[...]
    h3 = jnp.maximum(dis * (p_ref[0] + p_ref[1] + u3_ref[...]) + b3_ref[...], 0.0)
    csum = jnp.sum(h3, axis=0, keepdims=True)

    @pl.when(i == 0)
    def _():
        acc_ref[...] = csum

    @pl.when(i > 0)
    def _():
        acc_ref[...] = acc_ref[...] + csum

    @pl.when(i == NBLK - 1)
    def _():
        g = acc_ref[...] * (1.0 / N)
        out_ref[...] = lax.dot_general(
            g, wp_ref[...], (((1,), (1,)), ((), ())),
            preferred_element_type=jnp.float32) + bp_ref[...]


def _k4(P3, u3, dis, b3, Wp, bp):
    return pl.pallas_call(
        _k4_body,
        grid=(NBLK,),
        in_specs=[
            _part_spec(D), _row_spec(D), _row_spec(1),
            _full_spec((1, 128)), _full_spec((64, 128)), _full_spec((1, 64)),
        ],
        out_specs=_full_spec((1, 64)),
        out_shape=jax.ShapeDtypeStruct((1, 64), jnp.float32),
        scratch_shapes=[pltpu.VMEM((1, 128), jnp.float32)],
    )(P3, u3, dis, b3, Wp, bp)


def kernel(x, edge_index, W1, b1, W2, b2, W3, b3, Wp, bp):
    out_dtype = jnp.result_type(x.dtype, W1.dtype)
    f32 = jnp.float32
    x = x.astype(f32)
    W1, b1, W2, b2 = W1.astype(f32), b1.astype(f32), W2.astype(f32), b2.astype(f32)
    W3, b3, Wp, bp = W3.astype(f32), b3.astype(f32), Wp.astype(f32), bp.astype(f32)
    ei = edge_index.astype(jnp.int32)
    pad = EPAD - E
    src = jnp.concatenate([ei[0], jnp.zeros((pad,), jnp.int32)]).reshape(EPAD // CH, CH)
    dst = jnp.concatenate([ei[1], jnp.full((pad,), N, jnp.int32)]).reshape(EPAD // CH, CH)

    zerD = jnp.zeros((CH, D), f32)
    oneD = jnp.ones((CH, D), f32)

    degp = _deg_kernel(dst, oneD, zerD)
    u1, dis = _k1(degp, x, W1)
    P1 = _agg_kernel(u1, src, dst, zerD)
    u2 = _k2(P1, u1, dis, b1.reshape(1, -1))
    P2 = _agg_kernel(u2, src, dst, zerD)
    u3 = _k3(P2, u2, dis, b2.reshape(1, -1), W2, W3)
    P3 = _agg_kernel(u3, src, dst, zerD)
    out = _k4(P3, u3, dis, b3.reshape(1, -1), Wp, bp.reshape(1, -1))
    return out[0].astype(out_dtype)
